# baseline (speedup 1.0000x reference)
import functools
import math

import jax
import jax.numpy as jnp
from jax import lax
from jax.experimental import pallas as pl
from jax.experimental.pallas import tpu as pltpu
from jax.experimental.pallas import tpu_sc as plsc

DIM = 128
NO = 10000
NNODE = 10000
E = 320000
ROW = 144
NROWS = 10112
NS = 16
K = 64
NB = 4
IB = 10
NBLK = 32
NCHUNK = NBLK * IB
EPT = NCHUNK * K
ROWS_PER_TILE = NROWS // NS
_INV_SQRT_D = 1.0 / math.sqrt(DIM)


def _tables_body(ox, xw, xt,
                 wq_w, bq_w, wk_w, bk_w, wv_w, bv_w, wo_w,
                 wq_t, bq_t, wk_t, bk_t, wv_t, bv_t, wo_t,
                 tw_ref, tt_ref):
    o = ox[...]
    cw = xw[...]
    ct = xt[...]
    col = lax.broadcasted_iota(jnp.int32, (NO, 16), 1)

    def one(wq, bq, wk, bk, wv, bv, wo, out_ref):
        Wq = wq[...]
        Wk = wk[...]
        Wv = wv[...]
        Wo = wo[...]
        f32 = jnp.float32
        cq = (jnp.dot(cw, Wq[0:DIM], preferred_element_type=f32)
              + jnp.dot(ct, Wq[DIM:2 * DIM], preferred_element_type=f32)
              + bq[...])
        ck = (jnp.dot(cw, Wk[0:DIM], preferred_element_type=f32)
              + jnp.dot(ct, Wk[DIM:2 * DIM], preferred_element_type=f32)
              + bk[...])
        q = jnp.dot(o, Wq[2 * DIM:3 * DIM], preferred_element_type=f32) + cq
        k = jnp.dot(o, Wk[2 * DIM:3 * DIM], preferred_element_type=f32) + ck
        s = jnp.sum(q * k, axis=1, keepdims=True) * _INV_SQRT_D
        g = jnp.exp(s - jnp.max(s))
        cv = (jnp.dot(cw, Wv[0:DIM], preferred_element_type=f32)
              + jnp.dot(ct, Wv[DIM:2 * DIM], preferred_element_type=f32)
              + bv[...])
        cu = jnp.dot(cv, Wo, preferred_element_type=f32)
        M = jnp.dot(Wv[2 * DIM:3 * DIM], Wo, preferred_element_type=f32)
        u = jnp.dot(o, M, preferred_element_type=f32) + cu
        out_ref[:, 0:DIM] = g * u
        out_ref[:, DIM:ROW] = jnp.where(col == 0, g,
                                        jnp.where(col == 1, 1.0, 0.0))

    one(wq_w, bq_w, wk_w, bk_w, wv_w, bv_w, wo_w, tw_ref)
    one(wq_t, bq_t, wk_t, bk_t, wv_t, bv_t, wo_t, tt_ref)


def _build_tables(ox, xw, xt, args_w, args_t):
    out_sd = jax.ShapeDtypeStruct((NO, ROW), jnp.float32)
    return pl.pallas_call(
        _tables_body,
        out_shape=[out_sd, out_sd],
    )(ox, xw, xt, *args_w, *args_t)


def _edge_body(oidx_hbm, widx_hbm, tidx_hbm, tab_w_hbm, tab_t_hbm, zeros_hbm,
               out_w_hbm, out_t_hbm,
               oidx_v, didx_v, bufs, acc, gsems, ssems, isems):
    c = lax.axis_index("c")
    s = lax.axis_index("s")

    @pl.when(s == 0)
    def _():
        pltpu.sync_copy(zeros_hbm, acc)

    def run(didx_hbm, tab_hbm, out_hbm):
        def fire_refill(b, par):
            pltpu.async_copy(oidx_hbm.at[s, pl.ds(b * IB, IB)],
                             oidx_v[par], isems[par])
            pltpu.async_copy(didx_hbm.at[s, pl.ds(b * IB, IB)],
                             didx_v[par], isems[par])

        def wait_refill(b, par):
            pltpu.make_async_copy(oidx_hbm.at[s, pl.ds(b * IB, IB)],
                                  oidx_v[par], isems[par]).wait()
            pltpu.make_async_copy(didx_hbm.at[s, pl.ds(b * IB, IB)],
                                  didx_v[par], isems[par]).wait()

        def do_block(par):
            ov, dv = oidx_v[par], didx_v[par]
            gd = [None] * IB
            sd = [None] * IB

            def fire_gather(j):
                gd[j] = pltpu.async_copy(tab_hbm.at[ov.at[j]],
                                         bufs[j % NB], gsems[j % NB])

            for j in range(NB - 1):
                fire_gather(j)
            for j in range(IB):
                p = j % NB
                gd[j].wait()
                if j + NB - 1 < IB:
                    if j >= 1:
                        sd[j - 1].wait()
                    fire_gather(j + NB - 1)
                sd[j] = pltpu.async_copy(bufs[p], acc.at[dv.at[j]],
                                         ssems[p], add=True)
            for j in range(max(IB - NB, 0), IB):
                sd[j].wait()

        plsc.subcore_barrier()
        fire_refill(0, 0)

        @pl.loop(0, NBLK // 2)
        def _pair(g):
            b0 = 2 * g
            wait_refill(b0, 0)
            fire_refill(b0 + 1, 1)
            do_block(0)
            wait_refill(b0 + 1, 1)

            @pl.when(b0 + 2 < NBLK)
            def _():
                fire_refill(b0 + 2, 0)

            do_block(1)

        plsc.subcore_barrier()
        rbase = s * ROWS_PER_TILE
        pltpu.sync_copy(acc.at[pl.ds(rbase, ROWS_PER_TILE)],
                        out_hbm.at[pl.ds(rbase, ROWS_PER_TILE)])

    @pl.when(c == 0)
    def _():
        run(widx_hbm, tab_w_hbm, out_w_hbm)

    @pl.when(c == 1)
    def _():
        run(tidx_hbm, tab_t_hbm, out_t_hbm)


def _edge_scatter(oidx, widx, tidx, tab_w, tab_t, zeros):
    mesh = plsc.VectorSubcoreMesh(core_axis_name="c", subcore_axis_name="s")
    acc_sd = jax.ShapeDtypeStruct((NROWS, ROW), jnp.float32)
    fn = pl.kernel(
        _edge_body,
        out_type=[acc_sd, acc_sd],
        mesh=mesh,
        scratch_types=[
            [pltpu.VMEM((IB, K), jnp.int32) for _ in range(2)],
            [pltpu.VMEM((IB, K), jnp.int32) for _ in range(2)],
            [pltpu.VMEM((K, ROW), jnp.float32) for _ in range(NB)],
            pltpu.VMEM_SHARED((NROWS, ROW), jnp.float32),
            [pltpu.SemaphoreType.DMA for _ in range(NB)],
            [pltpu.SemaphoreType.DMA for _ in range(NB)],
            [pltpu.SemaphoreType.DMA for _ in range(2)],
        ],
        compiler_params=pltpu.CompilerParams(use_tc_tiling_on_sc=False),
    )
    return fn(oidx, widx, tidx, tab_w, tab_t, zeros)


def _finalize_body(aw, at, xw, xt, bo_w, bo_t, ln_g, ln_b, zw_ref, zt_ref):
    def one(acc_ref, x, bo, out_ref):
        P = acc_ref[0:NNODE, 0:DIM]
        dc = acc_ref[0:NNODE, DIM:ROW]
        D = dc[:, 0:1]
        C = dc[:, 1:2]
        h = x[...] + P / (D + 1e-16) + C * bo[...]
        mu = jnp.mean(h, axis=1, keepdims=True)
        hc = h - mu
        var = jnp.mean(hc * hc, axis=1, keepdims=True)
        out_ref[...] = hc * lax.rsqrt(var + 1e-5) * ln_g[...] + ln_b[...]

    one(aw, xw, bo_w, zw_ref)
    one(at, xt, bo_t, zt_ref)


def _finalize(acc_w, acc_t, xw, xt, bo_w, bo_t, ln_g, ln_b):
    out_sd = jax.ShapeDtypeStruct((NNODE, DIM), jnp.float32)
    return pl.pallas_call(
        _finalize_body,
        out_shape=[out_sd, out_sd],
    )(acc_w, acc_t, xw, xt, bo_w, bo_t, ln_g, ln_b)


def kernel(triple, option_x, num_worker, num_task, x_worker, x_task,
           Wq_w, bq_w, Wk_w, bk_w, Wv_w, bv_w, Wo_w, bo_w,
           Wq_t, bq_t, Wk_t, bk_t, Wv_t, bv_t, Wo_t, bo_t, ln_g, ln_b):
    xw = x_worker.reshape(1, DIM)
    xt = x_task.reshape(1, DIM)
    tab_w, tab_t = _build_tables(
        option_x, xw, xt,
        (Wq_w, bq_w.reshape(1, -1), Wk_w, bk_w.reshape(1, -1),
         Wv_w, bv_w.reshape(1, -1), Wo_w),
        (Wq_t, bq_t.reshape(1, -1), Wk_t, bk_t.reshape(1, -1),
         Wv_t, bv_t.reshape(1, -1), Wo_t))

    w_ids = triple[0] + (num_worker - NNODE)
    t_ids = triple[2] + (num_task - NNODE)
    o_ids = triple[1]

    def shape_idx(ids, pad):
        a = ids.reshape(NS, E // NS)
        a = jnp.pad(a, ((0, 0), (0, EPT - E // NS)), constant_values=pad)
        return a.reshape(NS, NCHUNK, K).astype(jnp.int32)

    oidx = shape_idx(o_ids, 0)
    widx = shape_idx(w_ids, NNODE)
    tidx = shape_idx(t_ids, NNODE)
    zeros = jnp.zeros((NROWS, ROW), jnp.float32)

    acc_w, acc_t = _edge_scatter(oidx, widx, tidx, tab_w, tab_t, zeros)

    z_w, z_t = _finalize(acc_w, acc_t, xw, xt,
                         bo_w.reshape(1, -1), bo_t.reshape(1, -1),
                         ln_g.reshape(1, -1), ln_b.reshape(1, -1))
    return (z_w, z_t, option_x)


# speedup vs baseline: 1.0072x; 1.0072x over previous
import functools
import math

import jax
import jax.numpy as jnp
from jax import lax
from jax.experimental import pallas as pl
from jax.experimental.pallas import tpu as pltpu
from jax.experimental.pallas import tpu_sc as plsc

DIM = 128
NO = 10000
NNODE = 10000
E = 320000
ROW = 144
NROWS = 10112
NS = 16
K = 64
NB = 4
IB = 20
NBLK = 16
NCHUNK = NBLK * IB
EPT = NCHUNK * K
ROWS_PER_TILE = NROWS // NS
_INV_SQRT_D = 1.0 / math.sqrt(DIM)


def _tables_body(ox, xw, xt,
                 wq_w, bq_w, wk_w, bk_w, wv_w, bv_w, wo_w,
                 wq_t, bq_t, wk_t, bk_t, wv_t, bv_t, wo_t,
                 tw_ref, tt_ref):
    o = ox[...]
    cw = xw[...]
    ct = xt[...]
    col = lax.broadcasted_iota(jnp.int32, (NO, 16), 1)

    def one(wq, bq, wk, bk, wv, bv, wo, out_ref):
        Wq = wq[...]
        Wk = wk[...]
        Wv = wv[...]
        Wo = wo[...]
        f32 = jnp.float32
        cq = (jnp.dot(cw, Wq[0:DIM], preferred_element_type=f32)
              + jnp.dot(ct, Wq[DIM:2 * DIM], preferred_element_type=f32)
              + bq[...])
        ck = (jnp.dot(cw, Wk[0:DIM], preferred_element_type=f32)
              + jnp.dot(ct, Wk[DIM:2 * DIM], preferred_element_type=f32)
              + bk[...])
        q = jnp.dot(o, Wq[2 * DIM:3 * DIM], preferred_element_type=f32) + cq
        k = jnp.dot(o, Wk[2 * DIM:3 * DIM], preferred_element_type=f32) + ck
        s = jnp.sum(q * k, axis=1, keepdims=True) * _INV_SQRT_D
        g = jnp.exp(s - jnp.max(s))
        cv = (jnp.dot(cw, Wv[0:DIM], preferred_element_type=f32)
              + jnp.dot(ct, Wv[DIM:2 * DIM], preferred_element_type=f32)
              + bv[...])
        cu = jnp.dot(cv, Wo, preferred_element_type=f32)
        M = jnp.dot(Wv[2 * DIM:3 * DIM], Wo, preferred_element_type=f32)
        u = jnp.dot(o, M, preferred_element_type=f32) + cu
        out_ref[:, 0:DIM] = g * u
        out_ref[:, DIM:ROW] = jnp.where(col == 0, g,
                                        jnp.where(col == 1, 1.0, 0.0))

    one(wq_w, bq_w, wk_w, bk_w, wv_w, bv_w, wo_w, tw_ref)
    one(wq_t, bq_t, wk_t, bk_t, wv_t, bv_t, wo_t, tt_ref)


def _build_tables(ox, xw, xt, args_w, args_t):
    out_sd = jax.ShapeDtypeStruct((NO, ROW), jnp.float32)
    return pl.pallas_call(
        _tables_body,
        out_shape=[out_sd, out_sd],
    )(ox, xw, xt, *args_w, *args_t)


def _edge_body(oidx_hbm, widx_hbm, tidx_hbm, tab_w_hbm, tab_t_hbm, zeros_hbm,
               out_w_hbm, out_t_hbm,
               oidx_v, didx_v, bufs, acc, gsems, ssems):
    c = lax.axis_index("c")
    s = lax.axis_index("s")

    @pl.when(s == 0)
    def _():
        pltpu.sync_copy(zeros_hbm, acc)

    def run(didx_hbm, tab_hbm, out_hbm):
        plsc.subcore_barrier()

        @pl.loop(0, NBLK)
        def _blk(b):
            pltpu.sync_copy(oidx_hbm.at[s, pl.ds(b * IB, IB)], oidx_v)
            pltpu.sync_copy(didx_hbm.at[s, pl.ds(b * IB, IB)], didx_v)
            gd = [None] * IB
            sd = [None] * IB

            def fire_gather(j):
                gd[j] = pltpu.async_copy(tab_hbm.at[oidx_v.at[j]],
                                         bufs[j % NB], gsems[j % NB])

            for j in range(NB - 1):
                fire_gather(j)
            for j in range(IB):
                p = j % NB
                gd[j].wait()
                if j + NB - 1 < IB:
                    if j >= 1:
                        sd[j - 1].wait()
                    fire_gather(j + NB - 1)
                sd[j] = pltpu.async_copy(bufs[p], acc.at[didx_v.at[j]],
                                         ssems[p], add=True)
            for j in range(max(IB - NB, 0), IB):
                sd[j].wait()

        plsc.subcore_barrier()
        rbase = s * ROWS_PER_TILE
        pltpu.sync_copy(acc.at[pl.ds(rbase, ROWS_PER_TILE)],
                        out_hbm.at[pl.ds(rbase, ROWS_PER_TILE)])

    @pl.when(c == 0)
    def _():
        run(widx_hbm, tab_w_hbm, out_w_hbm)

    @pl.when(c == 1)
    def _():
        run(tidx_hbm, tab_t_hbm, out_t_hbm)


def _edge_scatter(oidx, widx, tidx, tab_w, tab_t, zeros):
    mesh = plsc.VectorSubcoreMesh(core_axis_name="c", subcore_axis_name="s")
    acc_sd = jax.ShapeDtypeStruct((NROWS, ROW), jnp.float32)
    fn = pl.kernel(
        _edge_body,
        out_type=[acc_sd, acc_sd],
        mesh=mesh,
        scratch_types=[
            pltpu.VMEM((IB, K), jnp.int32),
            pltpu.VMEM((IB, K), jnp.int32),
            [pltpu.VMEM((K, ROW), jnp.float32) for _ in range(NB)],
            pltpu.VMEM_SHARED((NROWS, ROW), jnp.float32),
            [pltpu.SemaphoreType.DMA for _ in range(NB)],
            [pltpu.SemaphoreType.DMA for _ in range(NB)],
        ],
        compiler_params=pltpu.CompilerParams(use_tc_tiling_on_sc=False),
    )
    return fn(oidx, widx, tidx, tab_w, tab_t, zeros)


def _finalize_body(aw, at, xw, xt, bo_w, bo_t, ln_g, ln_b, zw_ref, zt_ref):
    def one(acc_ref, x, bo, out_ref):
        P = acc_ref[0:NNODE, 0:DIM]
        dc = acc_ref[0:NNODE, DIM:ROW]
        D = dc[:, 0:1]
        C = dc[:, 1:2]
        h = x[...] + P / (D + 1e-16) + C * bo[...]
        mu = jnp.mean(h, axis=1, keepdims=True)
        hc = h - mu
        var = jnp.mean(hc * hc, axis=1, keepdims=True)
        out_ref[...] = hc * lax.rsqrt(var + 1e-5) * ln_g[...] + ln_b[...]

    one(aw, xw, bo_w, zw_ref)
    one(at, xt, bo_t, zt_ref)


def _finalize(acc_w, acc_t, xw, xt, bo_w, bo_t, ln_g, ln_b):
    out_sd = jax.ShapeDtypeStruct((NNODE, DIM), jnp.float32)
    return pl.pallas_call(
        _finalize_body,
        out_shape=[out_sd, out_sd],
    )(acc_w, acc_t, xw, xt, bo_w, bo_t, ln_g, ln_b)


def kernel(triple, option_x, num_worker, num_task, x_worker, x_task,
           Wq_w, bq_w, Wk_w, bk_w, Wv_w, bv_w, Wo_w, bo_w,
           Wq_t, bq_t, Wk_t, bk_t, Wv_t, bv_t, Wo_t, bo_t, ln_g, ln_b):
    xw = x_worker.reshape(1, DIM)
    xt = x_task.reshape(1, DIM)
    tab_w, tab_t = _build_tables(
        option_x, xw, xt,
        (Wq_w, bq_w.reshape(1, -1), Wk_w, bk_w.reshape(1, -1),
         Wv_w, bv_w.reshape(1, -1), Wo_w),
        (Wq_t, bq_t.reshape(1, -1), Wk_t, bk_t.reshape(1, -1),
         Wv_t, bv_t.reshape(1, -1), Wo_t))

    w_ids = triple[0] + (num_worker - NNODE)
    t_ids = triple[2] + (num_task - NNODE)
    o_ids = triple[1]

    def shape_idx(ids, pad):
        a = ids.reshape(NS, E // NS)
        a = jnp.pad(a, ((0, 0), (0, EPT - E // NS)), constant_values=pad)
        return a.reshape(NS, NCHUNK, K).astype(jnp.int32)

    oidx = shape_idx(o_ids, 0)
    widx = shape_idx(w_ids, NNODE)
    tidx = shape_idx(t_ids, NNODE)
    zeros = jnp.zeros((NROWS, ROW), jnp.float32)

    acc_w, acc_t = _edge_scatter(oidx, widx, tidx, tab_w, tab_t, zeros)

    z_w, z_t = _finalize(acc_w, acc_t, xw, xt,
                         bo_w.reshape(1, -1), bo_t.reshape(1, -1),
                         ln_g.reshape(1, -1), ln_b.reshape(1, -1))
    return (z_w, z_t, option_x)


# speedup vs baseline: 1.7763x; 1.7636x over previous
import functools
import math

import jax
import jax.numpy as jnp
from jax import lax
from jax.experimental import pallas as pl
from jax.experimental.pallas import tpu as pltpu
from jax.experimental.pallas import tpu_sc as plsc

DIM = 128
NO = 10000
NNODE = 10000
E = 320000
ROW = 144
NROWS = 10112
NS = 16
K = 80
NB = 3
IB = 10
NBLK = 25
NCHUNK = NBLK * IB
EPT = NCHUNK * K
ROWS_PER_TILE = NROWS // NS
_INV_SQRT_D = 1.0 / math.sqrt(DIM)


def _tables_body(ox, xw, xt,
                 wq_w, bq_w, wk_w, bk_w, wv_w, bv_w, wo_w,
                 wq_t, bq_t, wk_t, bk_t, wv_t, bv_t, wo_t,
                 tw_ref, tt_ref):
    o = ox[...]
    cw = xw[...]
    ct = xt[...]
    col = lax.broadcasted_iota(jnp.int32, (NO, 16), 1)

    def one(wq, bq, wk, bk, wv, bv, wo, out_ref):
        Wq = wq[...]
        Wk = wk[...]
        Wv = wv[...]
        Wo = wo[...]
        f32 = jnp.float32
        cq = (jnp.dot(cw, Wq[0:DIM], preferred_element_type=f32)
              + jnp.dot(ct, Wq[DIM:2 * DIM], preferred_element_type=f32)
              + bq[...])
        ck = (jnp.dot(cw, Wk[0:DIM], preferred_element_type=f32)
              + jnp.dot(ct, Wk[DIM:2 * DIM], preferred_element_type=f32)
              + bk[...])
        q = jnp.dot(o, Wq[2 * DIM:3 * DIM], preferred_element_type=f32) + cq
        k = jnp.dot(o, Wk[2 * DIM:3 * DIM], preferred_element_type=f32) + ck
        s = jnp.sum(q * k, axis=1, keepdims=True) * _INV_SQRT_D
        g = jnp.exp(s - jnp.max(s))
        cv = (jnp.dot(cw, Wv[0:DIM], preferred_element_type=f32)
              + jnp.dot(ct, Wv[DIM:2 * DIM], preferred_element_type=f32)
              + bv[...])
        cu = jnp.dot(cv, Wo, preferred_element_type=f32)
        M = jnp.dot(Wv[2 * DIM:3 * DIM], Wo, preferred_element_type=f32)
        u = jnp.dot(o, M, preferred_element_type=f32) + cu
        out_ref[:, 0:DIM] = g * u
        out_ref[:, DIM:ROW] = jnp.where(col == 0, g,
                                        jnp.where(col == 1, 1.0, 0.0))

    one(wq_w, bq_w, wk_w, bk_w, wv_w, bv_w, wo_w, tw_ref)
    one(wq_t, bq_t, wk_t, bk_t, wv_t, bv_t, wo_t, tt_ref)


def _build_tables(ox, xw, xt, args_w, args_t):
    out_sd = jax.ShapeDtypeStruct((NO, ROW), jnp.float32)
    return pl.pallas_call(
        _tables_body,
        out_shape=[out_sd, out_sd],
    )(ox, xw, xt, *args_w, *args_t)


def _edge_body(oidx_hbm, widx_hbm, tidx_hbm, tab_w_hbm, tab_t_hbm, zeros_hbm,
               out_w_hbm, out_t_hbm,
               oidx_v, didx_v, bufs, acc, gsems, ssems):
    c = lax.axis_index("c")
    s = lax.axis_index("s")

    @pl.when(s == 0)
    def _():
        pltpu.sync_copy(zeros_hbm, acc)

    def run(didx_hbm, tab_hbm, out_hbm):
        plsc.subcore_barrier()

        @pl.loop(0, NBLK)
        def _blk(b):
            pltpu.sync_copy(oidx_hbm.at[s, pl.ds(b * IB, IB)], oidx_v)
            pltpu.sync_copy(didx_hbm.at[s, pl.ds(b * IB, IB)], didx_v)
            gd = [None] * IB
            sd = [None] * IB

            def fire_gather(j):
                gd[j] = pltpu.async_copy(tab_hbm.at[oidx_v.at[j]],
                                         bufs[j % NB], gsems[j % NB])

            for j in range(NB - 1):
                fire_gather(j)
            for j in range(IB):
                p = j % NB
                gd[j].wait()
                if j + NB - 1 < IB:
                    if j >= 1:
                        sd[j - 1].wait()
                    fire_gather(j + NB - 1)
                sd[j] = pltpu.async_copy(bufs[p], acc.at[didx_v.at[j]],
                                         ssems[p], add=True)
            for j in range(max(IB - NB, 0), IB):
                sd[j].wait()

        plsc.subcore_barrier()
        rbase = s * ROWS_PER_TILE
        pltpu.sync_copy(acc.at[pl.ds(rbase, ROWS_PER_TILE)],
                        out_hbm.at[pl.ds(rbase, ROWS_PER_TILE)])

    @pl.when(c == 0)
    def _():
        run(widx_hbm, tab_w_hbm, out_w_hbm)

    @pl.when(c == 1)
    def _():
        run(tidx_hbm, tab_t_hbm, out_t_hbm)


def _edge_scatter(oidx, widx, tidx, tab_w, tab_t, zeros):
    mesh = plsc.VectorSubcoreMesh(core_axis_name="c", subcore_axis_name="s")
    acc_sd = jax.ShapeDtypeStruct((NROWS, ROW), jnp.float32)
    fn = pl.kernel(
        _edge_body,
        out_type=[acc_sd, acc_sd],
        mesh=mesh,
        scratch_types=[
            pltpu.VMEM((IB, K), jnp.int32),
            pltpu.VMEM((IB, K), jnp.int32),
            [pltpu.VMEM((K, ROW), jnp.float32) for _ in range(NB)],
            pltpu.VMEM_SHARED((NROWS, ROW), jnp.float32),
            [pltpu.SemaphoreType.DMA for _ in range(NB)],
            [pltpu.SemaphoreType.DMA for _ in range(NB)],
        ],
        compiler_params=pltpu.CompilerParams(use_tc_tiling_on_sc=False),
    )
    return fn(oidx, widx, tidx, tab_w, tab_t, zeros)


def _finalize_body(aw, at, xw, xt, bo_w, bo_t, ln_g, ln_b, zw_ref, zt_ref):
    def one(acc_ref, x, bo, out_ref):
        P = acc_ref[0:NNODE, 0:DIM]
        dc = acc_ref[0:NNODE, DIM:ROW]
        D = dc[:, 0:1]
        C = dc[:, 1:2]
        h = x[...] + P / (D + 1e-16) + C * bo[...]
        mu = jnp.mean(h, axis=1, keepdims=True)
        hc = h - mu
        var = jnp.mean(hc * hc, axis=1, keepdims=True)
        out_ref[...] = hc * lax.rsqrt(var + 1e-5) * ln_g[...] + ln_b[...]

    one(aw, xw, bo_w, zw_ref)
    one(at, xt, bo_t, zt_ref)


def _finalize(acc_w, acc_t, xw, xt, bo_w, bo_t, ln_g, ln_b):
    out_sd = jax.ShapeDtypeStruct((NNODE, DIM), jnp.float32)
    return pl.pallas_call(
        _finalize_body,
        out_shape=[out_sd, out_sd],
    )(acc_w, acc_t, xw, xt, bo_w, bo_t, ln_g, ln_b)


def kernel(triple, option_x, num_worker, num_task, x_worker, x_task,
           Wq_w, bq_w, Wk_w, bk_w, Wv_w, bv_w, Wo_w, bo_w,
           Wq_t, bq_t, Wk_t, bk_t, Wv_t, bv_t, Wo_t, bo_t, ln_g, ln_b):
    xw = x_worker.reshape(1, DIM)
    xt = x_task.reshape(1, DIM)
    tab_w, tab_t = _build_tables(
        option_x, xw, xt,
        (Wq_w, bq_w.reshape(1, -1), Wk_w, bk_w.reshape(1, -1),
         Wv_w, bv_w.reshape(1, -1), Wo_w),
        (Wq_t, bq_t.reshape(1, -1), Wk_t, bk_t.reshape(1, -1),
         Wv_t, bv_t.reshape(1, -1), Wo_t))

    w_ids = triple[0] + (num_worker - NNODE)
    t_ids = triple[2] + (num_task - NNODE)
    o_ids = triple[1]

    def shape_idx(ids, pad):
        a = ids.reshape(NS, E // NS)
        a = jnp.pad(a, ((0, 0), (0, EPT - E // NS)), constant_values=pad)
        return a.reshape(NS, NCHUNK, K).astype(jnp.int32)

    oidx = shape_idx(o_ids, 0)
    widx = shape_idx(w_ids, NNODE)
    tidx = shape_idx(t_ids, NNODE)
    zeros = jnp.zeros((NROWS, ROW), jnp.float32)

    acc_w, acc_t = _edge_scatter(oidx, widx, tidx, tab_w, tab_t, zeros)

    z_w, z_t = _finalize(acc_w, acc_t, xw, xt,
                         bo_w.reshape(1, -1), bo_t.reshape(1, -1),
                         ln_g.reshape(1, -1), ln_b.reshape(1, -1))
    return (z_w, z_t, option_x)


# speedup vs baseline: 1.9351x; 1.0894x over previous
import functools
import math

import jax
import jax.numpy as jnp
from jax import lax
from jax.experimental import pallas as pl
from jax.experimental.pallas import tpu as pltpu
from jax.experimental.pallas import tpu_sc as plsc

DIM = 128
NO = 10000
NNODE = 10000
E = 320000
ROW = 144
NROWS = 10112
NS = 16
K = 80
NB = 3
IB = 25
NBLK = 10
NCHUNK = NBLK * IB
EPT = NCHUNK * K
ROWS_PER_TILE = NROWS // NS
_INV_SQRT_D = 1.0 / math.sqrt(DIM)


def _tables_body(ox, xw, xt,
                 wq_w, bq_w, wk_w, bk_w, wv_w, bv_w, wo_w,
                 wq_t, bq_t, wk_t, bk_t, wv_t, bv_t, wo_t,
                 tw_ref, tt_ref):
    o = ox[...]
    cw = xw[...]
    ct = xt[...]
    col = lax.broadcasted_iota(jnp.int32, (NO, 16), 1)

    def one(wq, bq, wk, bk, wv, bv, wo, out_ref):
        Wq = wq[...]
        Wk = wk[...]
        Wv = wv[...]
        Wo = wo[...]
        f32 = jnp.float32
        cq = (jnp.dot(cw, Wq[0:DIM], preferred_element_type=f32)
              + jnp.dot(ct, Wq[DIM:2 * DIM], preferred_element_type=f32)
              + bq[...])
        ck = (jnp.dot(cw, Wk[0:DIM], preferred_element_type=f32)
              + jnp.dot(ct, Wk[DIM:2 * DIM], preferred_element_type=f32)
              + bk[...])
        q = jnp.dot(o, Wq[2 * DIM:3 * DIM], preferred_element_type=f32) + cq
        k = jnp.dot(o, Wk[2 * DIM:3 * DIM], preferred_element_type=f32) + ck
        s = jnp.sum(q * k, axis=1, keepdims=True) * _INV_SQRT_D
        g = jnp.exp(s - jnp.max(s))
        cv = (jnp.dot(cw, Wv[0:DIM], preferred_element_type=f32)
              + jnp.dot(ct, Wv[DIM:2 * DIM], preferred_element_type=f32)
              + bv[...])
        cu = jnp.dot(cv, Wo, preferred_element_type=f32)
        M = jnp.dot(Wv[2 * DIM:3 * DIM], Wo, preferred_element_type=f32)
        u = jnp.dot(o, M, preferred_element_type=f32) + cu
        out_ref[:, 0:DIM] = g * u
        out_ref[:, DIM:ROW] = jnp.where(col == 0, g,
                                        jnp.where(col == 1, 1.0, 0.0))

    one(wq_w, bq_w, wk_w, bk_w, wv_w, bv_w, wo_w, tw_ref)
    one(wq_t, bq_t, wk_t, bk_t, wv_t, bv_t, wo_t, tt_ref)


def _build_tables(ox, xw, xt, args_w, args_t):
    out_sd = jax.ShapeDtypeStruct((NO, ROW), jnp.float32)
    return pl.pallas_call(
        _tables_body,
        out_shape=[out_sd, out_sd],
    )(ox, xw, xt, *args_w, *args_t)


def _edge_body(oidx_hbm, widx_hbm, tidx_hbm, tab_w_hbm, tab_t_hbm, zeros_hbm,
               out_w_hbm, out_t_hbm,
               oidx_v, didx_v, bufs, acc, gsems, ssems):
    c = lax.axis_index("c")
    s = lax.axis_index("s")

    @pl.when(s == 0)
    def _():
        pltpu.sync_copy(zeros_hbm, acc)

    def run(didx_hbm, tab_hbm, out_hbm):
        plsc.subcore_barrier()

        @pl.loop(0, NBLK)
        def _blk(b):
            pltpu.sync_copy(oidx_hbm.at[s, pl.ds(b * IB, IB)], oidx_v)
            pltpu.sync_copy(didx_hbm.at[s, pl.ds(b * IB, IB)], didx_v)
            gd = [None] * IB
            sd = [None] * IB

            def fire_gather(j):
                gd[j] = pltpu.async_copy(tab_hbm.at[oidx_v.at[j]],
                                         bufs[j % NB], gsems[j % NB])

            for j in range(NB - 1):
                fire_gather(j)
            for j in range(IB):
                p = j % NB
                gd[j].wait()
                if j + NB - 1 < IB:
                    if j >= 1:
                        sd[j - 1].wait()
                    fire_gather(j + NB - 1)
                sd[j] = pltpu.async_copy(bufs[p], acc.at[didx_v.at[j]],
                                         ssems[p], add=True)
            for j in range(max(IB - NB, 0), IB):
                sd[j].wait()

        plsc.subcore_barrier()
        rbase = s * ROWS_PER_TILE
        pltpu.sync_copy(acc.at[pl.ds(rbase, ROWS_PER_TILE)],
                        out_hbm.at[pl.ds(rbase, ROWS_PER_TILE)])

    @pl.when(c == 0)
    def _():
        run(widx_hbm, tab_w_hbm, out_w_hbm)

    @pl.when(c == 1)
    def _():
        run(tidx_hbm, tab_t_hbm, out_t_hbm)


def _edge_scatter(oidx, widx, tidx, tab_w, tab_t, zeros):
    mesh = plsc.VectorSubcoreMesh(core_axis_name="c", subcore_axis_name="s")
    acc_sd = jax.ShapeDtypeStruct((NROWS, ROW), jnp.float32)
    fn = pl.kernel(
        _edge_body,
        out_type=[acc_sd, acc_sd],
        mesh=mesh,
        scratch_types=[
            pltpu.VMEM((IB, K), jnp.int32),
            pltpu.VMEM((IB, K), jnp.int32),
            [pltpu.VMEM((K, ROW), jnp.float32) for _ in range(NB)],
            pltpu.VMEM_SHARED((NROWS, ROW), jnp.float32),
            [pltpu.SemaphoreType.DMA for _ in range(NB)],
            [pltpu.SemaphoreType.DMA for _ in range(NB)],
        ],
        compiler_params=pltpu.CompilerParams(use_tc_tiling_on_sc=False),
    )
    return fn(oidx, widx, tidx, tab_w, tab_t, zeros)


def _finalize_body(aw, at, xw, xt, bo_w, bo_t, ln_g, ln_b, zw_ref, zt_ref):
    def one(acc_ref, x, bo, out_ref):
        P = acc_ref[0:NNODE, 0:DIM]
        dc = acc_ref[0:NNODE, DIM:ROW]
        D = dc[:, 0:1]
        C = dc[:, 1:2]
        h = x[...] + P / (D + 1e-16) + C * bo[...]
        mu = jnp.mean(h, axis=1, keepdims=True)
        hc = h - mu
        var = jnp.mean(hc * hc, axis=1, keepdims=True)
        out_ref[...] = hc * lax.rsqrt(var + 1e-5) * ln_g[...] + ln_b[...]

    one(aw, xw, bo_w, zw_ref)
    one(at, xt, bo_t, zt_ref)


def _finalize(acc_w, acc_t, xw, xt, bo_w, bo_t, ln_g, ln_b):
    out_sd = jax.ShapeDtypeStruct((NNODE, DIM), jnp.float32)
    return pl.pallas_call(
        _finalize_body,
        out_shape=[out_sd, out_sd],
    )(acc_w, acc_t, xw, xt, bo_w, bo_t, ln_g, ln_b)


def kernel(triple, option_x, num_worker, num_task, x_worker, x_task,
           Wq_w, bq_w, Wk_w, bk_w, Wv_w, bv_w, Wo_w, bo_w,
           Wq_t, bq_t, Wk_t, bk_t, Wv_t, bv_t, Wo_t, bo_t, ln_g, ln_b):
    xw = x_worker.reshape(1, DIM)
    xt = x_task.reshape(1, DIM)
    tab_w, tab_t = _build_tables(
        option_x, xw, xt,
        (Wq_w, bq_w.reshape(1, -1), Wk_w, bk_w.reshape(1, -1),
         Wv_w, bv_w.reshape(1, -1), Wo_w),
        (Wq_t, bq_t.reshape(1, -1), Wk_t, bk_t.reshape(1, -1),
         Wv_t, bv_t.reshape(1, -1), Wo_t))

    w_ids = triple[0] + (num_worker - NNODE)
    t_ids = triple[2] + (num_task - NNODE)
    o_ids = triple[1]

    def shape_idx(ids, pad):
        a = ids.reshape(NS, E // NS)
        a = jnp.pad(a, ((0, 0), (0, EPT - E // NS)), constant_values=pad)
        return a.reshape(NS, NCHUNK, K).astype(jnp.int32)

    oidx = shape_idx(o_ids, 0)
    widx = shape_idx(w_ids, NNODE)
    tidx = shape_idx(t_ids, NNODE)
    zeros = jnp.zeros((NROWS, ROW), jnp.float32)

    acc_w, acc_t = _edge_scatter(oidx, widx, tidx, tab_w, tab_t, zeros)

    z_w, z_t = _finalize(acc_w, acc_t, xw, xt,
                         bo_w.reshape(1, -1), bo_t.reshape(1, -1),
                         ln_g.reshape(1, -1), ln_b.reshape(1, -1))
    return (z_w, z_t, option_x)
